# baseline (speedup 1.0000x reference)
import functools

import jax
import jax.numpy as jnp
from jax import lax
from jax.experimental import pallas as pl
from jax.experimental.pallas import tpu as pltpu
from jax.experimental.pallas import tpu_sc as plsc

B = 8192
V = 8192
D = 64
BR = 256
BC = 1024
NR = B // BR
NCT = V // BC
LOSS_SCALE = 1.25 / (B * D)

SC_CORES = 2
SC_SUBCORES = 16
NW = SC_CORES * SC_SUBCORES
BPW = B // NW
DP = 128
CHUNK = 128
NCH = BPW // CHUNK


def _argmin_body(x_ref, c_ref, ids_ref, loss_ref, run_val, run_idx, loss_acc):
    i = pl.program_id(0)
    j = pl.program_id(1)
    xb = x_ref[...].astype(jnp.bfloat16)
    cb = c_ref[...].astype(jnp.bfloat16)
    x = xb.astype(jnp.float32)
    c = cb.astype(jnp.float32)
    mm = lax.dot_general(xb, cb, (((1,), (1,)), ((), ())),
                         preferred_element_type=jnp.float32)
    cn = jnp.sum(c * c, axis=1)[None, :]
    xn = jnp.sum(x * x, axis=1, keepdims=True)
    dist = (xn + cn) - 2.0 * mm
    idx = lax.broadcasted_iota(jnp.int32, (BR, BC), 1) + j * BC

    @pl.when(j == 0)
    def _():
        run_val[...] = dist
        run_idx[...] = idx

    @pl.when(j > 0)
    def _():
        prev = run_val[...]
        pred = dist < prev
        run_val[...] = jnp.where(pred, dist, prev)
        run_idx[...] = jnp.where(pred, idx, run_idx[...])

    @pl.when(j == NCT - 1)
    def _():
        vals = run_val[...]
        m = jnp.min(vals, axis=1, keepdims=True)
        cand = jnp.where(vals == m, run_idx[...], jnp.int32(2**31 - 1))
        ids_ref[...] = jnp.min(cand, axis=1)
        part = jnp.sum(m)

        @pl.when(i == 0)
        def _():
            loss_acc[0] = part

        @pl.when(i > 0)
        def _():
            loss_acc[0] = loss_acc[0] + part

        @pl.when(i == NR - 1)
        def _():
            loss_ref[0] = loss_acc[0] * LOSS_SCALE


def _argmin_call(x, codebook):
    return pl.pallas_call(
        _argmin_body,
        grid=(NR, NCT),
        in_specs=[
            pl.BlockSpec((BR, D), lambda i, j: (i, 0)),
            pl.BlockSpec((BC, D), lambda i, j: (j, 0)),
        ],
        out_specs=[
            pl.BlockSpec((BR,), lambda i, j: (i,)),
            pl.BlockSpec((1,), lambda i, j: (0,), memory_space=pltpu.SMEM),
        ],
        out_shape=[
            jax.ShapeDtypeStruct((B,), jnp.int32),
            jax.ShapeDtypeStruct((1,), jnp.float32),
        ],
        scratch_shapes=[
            pltpu.VMEM((BR, BC), jnp.float32),
            pltpu.VMEM((BR, BC), jnp.int32),
            pltpu.SMEM((1,), jnp.float32),
        ],
    )(x, codebook)


@functools.lru_cache(maxsize=None)
def _sc_gather_fn():
    mesh = plsc.VectorSubcoreMesh(core_axis_name="c", subcore_axis_name="s")

    @functools.partial(
        pl.kernel,
        mesh=mesh,
        out_type=jax.ShapeDtypeStruct((B, DP), jnp.float32),
        scratch_types=[
            pltpu.VMEM((NCH, CHUNK), jnp.int32),
            pltpu.VMEM((NCH, CHUNK, DP), jnp.float32),
            pltpu.SemaphoreType.DMA,
        ],
    )
    def _gather(table_hbm, idx_hbm, out_hbm, idx_v, rows_v, sem):
        wid = lax.axis_index("s") * SC_CORES + lax.axis_index("c")
        base = wid * BPW
        for k in range(NCH):
            pltpu.sync_copy(idx_hbm.at[pl.ds(base + k * CHUNK, CHUNK)],
                            idx_v.at[k])
        copies = [
            pltpu.async_copy(table_hbm.at[idx_v.at[k]], rows_v.at[k], sem)
            for k in range(NCH)
        ]
        for cp in copies:
            cp.wait()
        for k in range(NCH):
            pltpu.sync_copy(rows_v.at[k],
                            out_hbm.at[pl.ds(base + k * CHUNK, CHUNK)])

    return _gather


def kernel(x, codebook, temperature):
    ids, loss = _argmin_call(x, codebook)
    cb_padded = jnp.concatenate(
        [codebook, jnp.zeros((V, DP - D), codebook.dtype)], axis=1)
    emb_out = _sc_gather_fn()(cb_padded, ids)[:, :D]
    return emb_out, ids, loss[0]


# speedup vs baseline: 1.5273x; 1.5273x over previous
import functools

import jax
import jax.numpy as jnp
from jax import lax
from jax.experimental import pallas as pl
from jax.experimental.pallas import tpu as pltpu
from jax.experimental.pallas import tpu_sc as plsc

B = 8192
V = 8192
D = 64
BR = 512
BC = 2048
NR = B // BR
NCT = V // BC
LOSS_SCALE = 1.25 / (B * D)

SC_CORES = 2
SC_SUBCORES = 16
NW = SC_CORES * SC_SUBCORES
BPW = B // NW
DP = 128
CHUNK = 128
NCH = BPW // CHUNK


def _argmin_body(x_ref, c_ref, ids_ref, loss_ref, run_val, run_idx, loss_acc):
    i = pl.program_id(0)
    j = pl.program_id(1)
    xb = x_ref[...].astype(jnp.bfloat16)
    cb = c_ref[...].astype(jnp.bfloat16)
    x = xb.astype(jnp.float32)
    c = cb.astype(jnp.float32)
    mm = lax.dot_general(xb, cb, (((1,), (1,)), ((), ())),
                         preferred_element_type=jnp.float32)
    cn = jnp.sum(c * c, axis=1)[None, :]
    xn = jnp.sum(x * x, axis=1, keepdims=True)
    dist = (xn + cn) - 2.0 * mm
    idx = lax.broadcasted_iota(jnp.int32, (BR, BC), 1) + j * BC

    @pl.when(j == 0)
    def _():
        run_val[...] = dist
        run_idx[...] = idx

    @pl.when(j > 0)
    def _():
        prev = run_val[...]
        pred = dist < prev
        run_val[...] = jnp.where(pred, dist, prev)
        run_idx[...] = jnp.where(pred, idx, run_idx[...])

    @pl.when(j == NCT - 1)
    def _():
        vals = run_val[...]
        m = jnp.min(vals, axis=1, keepdims=True)
        cand = jnp.where(vals == m, run_idx[...], jnp.int32(2**31 - 1))
        ids_ref[...] = jnp.min(cand, axis=1)
        part = jnp.sum(m)

        @pl.when(i == 0)
        def _():
            loss_acc[0] = part

        @pl.when(i > 0)
        def _():
            loss_acc[0] = loss_acc[0] + part

        @pl.when(i == NR - 1)
        def _():
            loss_ref[0] = loss_acc[0] * LOSS_SCALE


def _argmin_call(x, codebook):
    return pl.pallas_call(
        _argmin_body,
        grid=(NR, NCT),
        in_specs=[
            pl.BlockSpec((BR, D), lambda i, j: (i, 0)),
            pl.BlockSpec((BC, D), lambda i, j: (j, 0)),
        ],
        out_specs=[
            pl.BlockSpec((BR,), lambda i, j: (i,)),
            pl.BlockSpec((1,), lambda i, j: (0,), memory_space=pltpu.SMEM),
        ],
        out_shape=[
            jax.ShapeDtypeStruct((B,), jnp.int32),
            jax.ShapeDtypeStruct((1,), jnp.float32),
        ],
        scratch_shapes=[
            pltpu.VMEM((BR, BC), jnp.float32),
            pltpu.VMEM((BR, BC), jnp.int32),
            pltpu.SMEM((1,), jnp.float32),
        ],
    )(x, codebook)


@functools.lru_cache(maxsize=None)
def _sc_gather_fn():
    mesh = plsc.VectorSubcoreMesh(core_axis_name="c", subcore_axis_name="s")

    @functools.partial(
        pl.kernel,
        mesh=mesh,
        out_type=jax.ShapeDtypeStruct((B, DP), jnp.float32),
        scratch_types=[
            pltpu.VMEM((NCH, CHUNK), jnp.int32),
            pltpu.VMEM((NCH, CHUNK, DP), jnp.float32),
            pltpu.SemaphoreType.DMA,
        ],
    )
    def _gather(table_hbm, idx_hbm, out_hbm, idx_v, rows_v, sem):
        wid = lax.axis_index("s") * SC_CORES + lax.axis_index("c")
        base = wid * BPW
        for k in range(NCH):
            pltpu.sync_copy(idx_hbm.at[pl.ds(base + k * CHUNK, CHUNK)],
                            idx_v.at[k])
        copies = [
            pltpu.async_copy(table_hbm.at[idx_v.at[k]], rows_v.at[k], sem)
            for k in range(NCH)
        ]
        for cp in copies:
            cp.wait()
        for k in range(NCH):
            pltpu.sync_copy(rows_v.at[k],
                            out_hbm.at[pl.ds(base + k * CHUNK, CHUNK)])

    return _gather


def kernel(x, codebook, temperature):
    ids, loss = _argmin_call(x, codebook)
    cb_padded = jnp.concatenate(
        [codebook, jnp.zeros((V, DP - D), codebook.dtype)], axis=1)
    emb_out = _sc_gather_fn()(cb_padded, ids)[:, :D]
    return emb_out, ids, loss[0]
